# Initial kernel scaffold; baseline (speedup 1.0000x reference)
import jax
import jax.numpy as jnp
from jax.experimental import pallas as pl


def kernel(y_1, y_2, idx_label, edge_index, edge_weight):
    raise NotImplementedError("write your pallas kernel here")


# speedup vs baseline: 14.5458x; 14.5458x over previous
import functools

import jax
import jax.numpy as jnp
from jax import lax
from jax.experimental import pallas as pl
from jax.experimental.pallas import tpu as pltpu
from jax.experimental.pallas import tpu_sc as plsc

N = 10000
C = 128
E = 320000
NLAB = 5000

NC, NS, LANES = 2, 16, 16
N_PAD = 10240
K = 128
B_T = 157
PER_TILE = K * B_T
E_PAD = NS * PER_TILE
PAD_ROW = N
PAD_COL = N + 16
PAD_LAB = N + 32
L_PAD = 5120
ZR = 8
RB = 512
RB3 = 1024


def _stats_body(y_ref, s_ref, h_ref, lse_ref):
    g = pl.program_id(0)
    yb = y_ref[:]
    m = jnp.max(yb, axis=1, keepdims=True)
    e = jnp.exp(yb - m)
    se = jnp.sum(e, axis=1, keepdims=True)
    lse = m + jnp.log(se)
    sm = e / se
    hh = jnp.sum(sm * yb, axis=1, keepdims=True) - lse
    rowid = g * RB + lax.broadcasted_iota(jnp.int32, (RB, 1), 0)
    valid = (rowid < N).astype(jnp.float32)
    s_ref[:] = sm * valid
    h_ref[:] = hh * valid
    lse_ref[:] = lse * valid


def _node_stats(yp):
    return pl.pallas_call(
        _stats_body,
        grid=(N_PAD // RB,),
        in_specs=[pl.BlockSpec((RB, C), lambda g: (g, 0))],
        out_specs=[pl.BlockSpec((RB, C), lambda g: (g, 0)),
                   pl.BlockSpec((RB, 1), lambda g: (g, 0)),
                   pl.BlockSpec((RB, 1), lambda g: (g, 0))],
        out_shape=[jax.ShapeDtypeStruct((N_PAD, C), jnp.float32),
                   jax.ShapeDtypeStruct((N_PAD, 1), jnp.float32),
                   jax.ShapeDtypeStruct((N_PAD, 1), jnp.float32)],
    )(yp)


def _edge_body(s1, s2, h1, h2, pkp, labp,
               z1_out, z2_out, cs_out, hs1_out, hs2_out, lab_out,
               pk, rows, hloc, hsl, csl, zv, zsh, sem):
    c = lax.axis_index("c")
    s = lax.axis_index("s")
    zero16 = jnp.zeros((LANES,), jnp.float32)
    one16 = jnp.ones((LANES,), jnp.float32)

    @pl.when(jnp.logical_and(c == 0, s == 0))
    def _():
        for i in range(N_PAD // LANES):
            csl[pl.ds(i * LANES, LANES)] = zero16
        for i in range(L_PAD // K):
            pltpu.sync_copy(labp.at[pl.ds(i * K, K)], pk.at[0])
            for j in range(K // LANES):
                lab16 = pk[0, pl.ds(j * LANES, LANES)]
                plsc.addupdate_scatter(csl, [lab16], one16)
        pltpu.sync_copy(csl, lab_out)

    for i in range(ZR):
        for j in range(C // LANES):
            zv[i, pl.ds(j * LANES, LANES)] = zero16
    rows_per_tile = N_PAD // NS
    for b in range(rows_per_tile // ZR):
        pltpu.sync_copy(zv, zsh.at[pl.ds(s * rows_per_tile + b * ZR, ZR)])

    for i in range(N_PAD // LANES):
        hsl[pl.ds(i * LANES, LANES)] = zero16
        csl[pl.ds(i * LANES, LANES)] = zero16
    plsc.subcore_barrier()

    def run_edges(tbl, hbm_h, with_colsum):
        pltpu.sync_copy(hbm_h, hloc)

        def body(b, carry):
            pltpu.sync_copy(pkp.at[s].at[b], pk)
            pltpu.async_copy(tbl.at[pk.at[0]], rows, sem).wait()
            pltpu.sync_copy(rows, zsh.at[pk.at[1]], add=True)
            for j in range(K // LANES):
                row16 = pk[0, pl.ds(j * LANES, LANES)]
                col16 = pk[1, pl.ds(j * LANES, LANES)]
                h16 = plsc.load_gather(hloc, [row16])
                plsc.addupdate_scatter(hsl, [col16], h16)
                if with_colsum:
                    ew16 = plsc.bitcast(
                        pk[2, pl.ds(j * LANES, LANES)], jnp.float32)
                    plsc.addupdate_scatter(csl, [col16], ew16)
            return carry
        lax.fori_loop(0, B_T, body, 0)

    @pl.when(c == 0)
    def _():
        run_edges(s1, h1, True)
        pltpu.sync_copy(hsl, hs1_out.at[s])
        pltpu.sync_copy(csl, cs_out.at[s])

    @pl.when(c == 1)
    def _():
        run_edges(s2, h2, False)
        pltpu.sync_copy(hsl, hs2_out.at[s])

    plsc.subcore_barrier()

    sl = pl.ds(s * rows_per_tile, rows_per_tile)

    @pl.when(c == 0)
    def _():
        pltpu.sync_copy(zsh.at[sl], z1_out.at[sl])

    @pl.when(c == 1)
    def _():
        pltpu.sync_copy(zsh.at[sl], z2_out.at[sl])


@functools.cache
def _build_edge_stage():
    mesh = plsc.VectorSubcoreMesh(
        core_axis_name="c", subcore_axis_name="s",
        num_cores=NC, num_subcores=NS)
    return pl.kernel(
        _edge_body,
        out_type=(
            jax.ShapeDtypeStruct((N_PAD, C), jnp.float32),
            jax.ShapeDtypeStruct((N_PAD, C), jnp.float32),
            jax.ShapeDtypeStruct((NS, N_PAD), jnp.float32),
            jax.ShapeDtypeStruct((NS, N_PAD), jnp.float32),
            jax.ShapeDtypeStruct((NS, N_PAD), jnp.float32),
            jax.ShapeDtypeStruct((N_PAD,), jnp.float32),
        ),
        mesh=mesh,
        compiler_params=pltpu.CompilerParams(needs_layout_passes=False),
        scratch_types=[
            pltpu.VMEM((3, K), jnp.int32),
            pltpu.VMEM((K, C), jnp.float32),
            pltpu.VMEM((N_PAD,), jnp.float32),
            pltpu.VMEM((N_PAD,), jnp.float32),
            pltpu.VMEM((N_PAD,), jnp.float32),
            pltpu.VMEM((ZR, C), jnp.float32),
            pltpu.VMEM_SHARED((N_PAD, C), jnp.float32),
            pltpu.SemaphoreType.DMA,
        ],
    )


def _finish_body(y1_ref, y2_ref, lse1_ref, lse2_ref, z1_ref, z2_ref,
                 cs_ref, hs1_ref, hs2_ref, lc_ref, o_ref, acc):
    g = pl.program_id(0)

    @pl.when(g == 0)
    def _():
        for i in range(4):
            acc[i] = 0.0

    mask = (lc_ref[:] > 0.0).astype(jnp.float32)
    colsum = jnp.sum(cs_ref[:], axis=1, keepdims=True)
    indeg = jnp.sum(z1_ref[:], axis=1, keepdims=True)

    def kl_term(y_ref, lse_ref, z_ref, hs_ref):
        cross = jnp.sum(z_ref[:] * y_ref[:], axis=1, keepdims=True)
        hsum = jnp.sum(hs_ref[:], axis=1, keepdims=True)
        return jnp.sum(mask * (hsum + indeg * lse_ref[:] - cross))

    kl1 = kl_term(y1_ref, lse1_ref, z1_ref, hs1_ref)
    kl2 = kl_term(y2_ref, lse2_ref, z2_ref, hs2_ref)
    acc[0] = acc[0] + kl1
    acc[1] = acc[1] + kl2
    acc[2] = acc[2] + jnp.sum(mask * indeg)
    acc[3] = acc[3] + jnp.sum(mask * (colsum != 0.0).astype(jnp.float32))

    @pl.when(g == N_PAD // RB3 - 1)
    def _():
        loss = acc[3] * (acc[0] + acc[1]) / (acc[2] * float(NLAB))
        o_ref[:] = jnp.full((1, 1), loss, jnp.float32)


def _finish(y1p, y2p, lse1, lse2, z1, z2, cst, hs1t, hs2t, lc):
    nblk = N_PAD // RB3
    return pl.pallas_call(
        _finish_body,
        grid=(nblk,),
        in_specs=[
            pl.BlockSpec((RB3, C), lambda g: (g, 0)),
            pl.BlockSpec((RB3, C), lambda g: (g, 0)),
            pl.BlockSpec((RB3, 1), lambda g: (g, 0)),
            pl.BlockSpec((RB3, 1), lambda g: (g, 0)),
            pl.BlockSpec((RB3, C), lambda g: (g, 0)),
            pl.BlockSpec((RB3, C), lambda g: (g, 0)),
            pl.BlockSpec((RB3, NS), lambda g: (g, 0)),
            pl.BlockSpec((RB3, NS), lambda g: (g, 0)),
            pl.BlockSpec((RB3, NS), lambda g: (g, 0)),
            pl.BlockSpec((RB3, 1), lambda g: (g, 0)),
        ],
        out_specs=pl.BlockSpec((1, 1), lambda g: (0, 0)),
        out_shape=jax.ShapeDtypeStruct((1, 1), jnp.float32),
        scratch_shapes=[pltpu.SMEM((4,), jnp.float32)],
    )(y1p, y2p, lse1, lse2, z1, z2, cst, hs1t, hs2t, lc)


def kernel(y_1, y_2, idx_label, edge_index, edge_weight):
    y1p = jnp.pad(y_1, ((0, N_PAD - N), (0, 0)))
    y2p = jnp.pad(y_2, ((0, N_PAD - N), (0, 0)))
    s1, h1, lse1 = _node_stats(y1p)
    s2, h2, lse2 = _node_stats(y2p)
    pad_e = E_PAD - E
    row = jnp.concatenate(
        [edge_index[0], jnp.full((pad_e,), PAD_ROW, jnp.int32)])
    col = jnp.concatenate(
        [edge_index[1], jnp.full((pad_e,), PAD_COL, jnp.int32)])
    ewi = lax.bitcast_convert_type(
        jnp.concatenate([edge_weight, jnp.zeros((pad_e,), jnp.float32)]),
        jnp.int32)
    pk = jnp.stack([row, col, ewi]).reshape(3, NS, B_T, K).transpose(1, 2, 0, 3)
    lab = jnp.concatenate(
        [idx_label, jnp.full((L_PAD - NLAB,), PAD_LAB, jnp.int32)])
    z1, z2, cs, hs1, hs2, lc = _build_edge_stage()(
        s1, s2, h1.reshape(N_PAD), h2.reshape(N_PAD), pk, lab)
    loss = _finish(y1p, y2p, lse1, lse2, z1, z2,
                   cs.T, hs1.T, hs2.T, lc.reshape(N_PAD, 1))
    return loss.reshape(())


# speedup vs baseline: 15.8070x; 1.0867x over previous
import functools

import jax
import jax.numpy as jnp
from jax import lax
from jax.experimental import pallas as pl
from jax.experimental.pallas import tpu as pltpu
from jax.experimental.pallas import tpu_sc as plsc

N = 10000
C = 128
E = 320000
NLAB = 5000

NC, NS, LANES = 2, 16, 16
N_PAD = 10240
K = 64
B_T = 314
PER_TILE = K * B_T
E_PAD = NS * PER_TILE
PAD_ROW = N
PAD_COL = N + 16
PAD_LAB = N + 32
L_PAD = 5120
ZR = 8
RB = 512
RB3 = 1024


def _stats_body(y_ref, s_ref, h_ref, lse_ref):
    g = pl.program_id(0)
    yb = y_ref[:]
    m = jnp.max(yb, axis=1, keepdims=True)
    e = jnp.exp(yb - m)
    se = jnp.sum(e, axis=1, keepdims=True)
    lse = m + jnp.log(se)
    sm = e / se
    hh = jnp.sum(sm * yb, axis=1, keepdims=True) - lse
    rowid = g * RB + lax.broadcasted_iota(jnp.int32, (RB, 1), 0)
    valid = (rowid < N).astype(jnp.float32)
    s_ref[:] = sm * valid
    h_ref[:] = hh * valid
    lse_ref[:] = lse * valid


def _node_stats(yp):
    return pl.pallas_call(
        _stats_body,
        grid=(N_PAD // RB,),
        in_specs=[pl.BlockSpec((RB, C), lambda g: (g, 0))],
        out_specs=[pl.BlockSpec((RB, C), lambda g: (g, 0)),
                   pl.BlockSpec((RB, 1), lambda g: (g, 0)),
                   pl.BlockSpec((RB, 1), lambda g: (g, 0))],
        out_shape=[jax.ShapeDtypeStruct((N_PAD, C), jnp.float32),
                   jax.ShapeDtypeStruct((N_PAD, 1), jnp.float32),
                   jax.ShapeDtypeStruct((N_PAD, 1), jnp.float32)],
    )(yp)


def _edge_body(s1, s2, h1, h2, pkp, labp,
               z1_out, z2_out, cs_out, hs1_out, hs2_out, lab_out,
               pk0, pk1, rows0, rows1, hloc, hsl, csl, zv, zsh, sem0, sem1):
    c = lax.axis_index("c")
    s = lax.axis_index("s")
    zero16 = jnp.zeros((LANES,), jnp.float32)
    one16 = jnp.ones((LANES,), jnp.float32)

    @pl.when(jnp.logical_and(c == 0, s == 0))
    def _():
        for i in range(N_PAD // LANES):
            csl[pl.ds(i * LANES, LANES)] = zero16
        for i in range(L_PAD // K):
            pltpu.sync_copy(labp.at[pl.ds(i * K, K)], pk0.at[0])
            for j in range(K // LANES):
                lab16 = pk0[0, pl.ds(j * LANES, LANES)]
                plsc.addupdate_scatter(csl, [lab16], one16)
        pltpu.sync_copy(csl, lab_out)

    for i in range(ZR):
        for j in range(C // LANES):
            zv[i, pl.ds(j * LANES, LANES)] = zero16
    rows_per_tile = N_PAD // NS
    for b in range(rows_per_tile // ZR):
        pltpu.sync_copy(zv, zsh.at[pl.ds(s * rows_per_tile + b * ZR, ZR)])

    for i in range(N_PAD // LANES):
        hsl[pl.ds(i * LANES, LANES)] = zero16
        csl[pl.ds(i * LANES, LANES)] = zero16
    plsc.subcore_barrier()

    def run_edges(tbl, hbm_h, with_colsum):
        pltpu.sync_copy(hbm_h, hloc)

        def proc(pk, rows, sem):
            pltpu.make_async_copy(tbl.at[pk.at[0]], rows, sem).wait()
            pltpu.sync_copy(rows, zsh.at[pk.at[1]], add=True)
            for j in range(K // LANES):
                row16 = pk[0, pl.ds(j * LANES, LANES)]
                col16 = pk[1, pl.ds(j * LANES, LANES)]
                h16 = plsc.load_gather(hloc, [row16])
                plsc.addupdate_scatter(hsl, [col16], h16)
                if with_colsum:
                    ew16 = plsc.bitcast(
                        pk[2, pl.ds(j * LANES, LANES)], jnp.float32)
                    plsc.addupdate_scatter(csl, [col16], ew16)

        def fetch(b, pk, rows, sem):
            pltpu.sync_copy(pkp.at[s].at[b], pk)
            pltpu.async_copy(tbl.at[pk.at[0]], rows, sem)

        fetch(0, pk0, rows0, sem0)

        def pair(p, carry):
            fetch(2 * p + 1, pk1, rows1, sem1)
            proc(pk0, rows0, sem0)

            @pl.when(2 * p + 2 < B_T)
            def _():
                fetch(2 * p + 2, pk0, rows0, sem0)
            proc(pk1, rows1, sem1)
            return carry
        lax.fori_loop(0, B_T // 2, pair, 0)

    @pl.when(c == 0)
    def _():
        run_edges(s1, h1, True)
        pltpu.sync_copy(hsl, hs1_out.at[s])
        pltpu.sync_copy(csl, cs_out.at[s])

    @pl.when(c == 1)
    def _():
        run_edges(s2, h2, False)
        pltpu.sync_copy(hsl, hs2_out.at[s])

    plsc.subcore_barrier()

    sl = pl.ds(s * rows_per_tile, rows_per_tile)

    @pl.when(c == 0)
    def _():
        pltpu.sync_copy(zsh.at[sl], z1_out.at[sl])

    @pl.when(c == 1)
    def _():
        pltpu.sync_copy(zsh.at[sl], z2_out.at[sl])


@functools.cache
def _build_edge_stage():
    mesh = plsc.VectorSubcoreMesh(
        core_axis_name="c", subcore_axis_name="s",
        num_cores=NC, num_subcores=NS)
    return pl.kernel(
        _edge_body,
        out_type=(
            jax.ShapeDtypeStruct((N_PAD, C), jnp.float32),
            jax.ShapeDtypeStruct((N_PAD, C), jnp.float32),
            jax.ShapeDtypeStruct((NS, N_PAD), jnp.float32),
            jax.ShapeDtypeStruct((NS, N_PAD), jnp.float32),
            jax.ShapeDtypeStruct((NS, N_PAD), jnp.float32),
            jax.ShapeDtypeStruct((N_PAD,), jnp.float32),
        ),
        mesh=mesh,
        compiler_params=pltpu.CompilerParams(needs_layout_passes=False),
        scratch_types=[
            pltpu.VMEM((3, K), jnp.int32),
            pltpu.VMEM((3, K), jnp.int32),
            pltpu.VMEM((K, C), jnp.float32),
            pltpu.VMEM((K, C), jnp.float32),
            pltpu.VMEM((N_PAD,), jnp.float32),
            pltpu.VMEM((N_PAD,), jnp.float32),
            pltpu.VMEM((N_PAD,), jnp.float32),
            pltpu.VMEM((ZR, C), jnp.float32),
            pltpu.VMEM_SHARED((N_PAD, C), jnp.float32),
            pltpu.SemaphoreType.DMA,
            pltpu.SemaphoreType.DMA,
        ],
    )


def _finish_body(y1_ref, y2_ref, lse1_ref, lse2_ref, z1_ref, z2_ref,
                 cs_ref, hs1_ref, hs2_ref, lc_ref, o_ref, acc):
    g = pl.program_id(0)

    @pl.when(g == 0)
    def _():
        for i in range(4):
            acc[i] = 0.0

    mask = (lc_ref[:] > 0.0).astype(jnp.float32)
    colsum = jnp.sum(cs_ref[:], axis=1, keepdims=True)
    indeg = jnp.sum(z1_ref[:], axis=1, keepdims=True)

    def kl_term(y_ref, lse_ref, z_ref, hs_ref):
        cross = jnp.sum(z_ref[:] * y_ref[:], axis=1, keepdims=True)
        hsum = jnp.sum(hs_ref[:], axis=1, keepdims=True)
        return jnp.sum(mask * (hsum + indeg * lse_ref[:] - cross))

    kl1 = kl_term(y1_ref, lse1_ref, z1_ref, hs1_ref)
    kl2 = kl_term(y2_ref, lse2_ref, z2_ref, hs2_ref)
    acc[0] = acc[0] + kl1
    acc[1] = acc[1] + kl2
    acc[2] = acc[2] + jnp.sum(mask * indeg)
    acc[3] = acc[3] + jnp.sum(mask * (colsum != 0.0).astype(jnp.float32))

    @pl.when(g == N_PAD // RB3 - 1)
    def _():
        loss = acc[3] * (acc[0] + acc[1]) / (acc[2] * float(NLAB))
        o_ref[:] = jnp.full((1, 1), loss, jnp.float32)


def _finish(y1p, y2p, lse1, lse2, z1, z2, cst, hs1t, hs2t, lc):
    nblk = N_PAD // RB3
    return pl.pallas_call(
        _finish_body,
        grid=(nblk,),
        in_specs=[
            pl.BlockSpec((RB3, C), lambda g: (g, 0)),
            pl.BlockSpec((RB3, C), lambda g: (g, 0)),
            pl.BlockSpec((RB3, 1), lambda g: (g, 0)),
            pl.BlockSpec((RB3, 1), lambda g: (g, 0)),
            pl.BlockSpec((RB3, C), lambda g: (g, 0)),
            pl.BlockSpec((RB3, C), lambda g: (g, 0)),
            pl.BlockSpec((RB3, NS), lambda g: (g, 0)),
            pl.BlockSpec((RB3, NS), lambda g: (g, 0)),
            pl.BlockSpec((RB3, NS), lambda g: (g, 0)),
            pl.BlockSpec((RB3, 1), lambda g: (g, 0)),
        ],
        out_specs=pl.BlockSpec((1, 1), lambda g: (0, 0)),
        out_shape=jax.ShapeDtypeStruct((1, 1), jnp.float32),
        scratch_shapes=[pltpu.SMEM((4,), jnp.float32)],
    )(y1p, y2p, lse1, lse2, z1, z2, cst, hs1t, hs2t, lc)


def kernel(y_1, y_2, idx_label, edge_index, edge_weight):
    y1p = jnp.pad(y_1, ((0, N_PAD - N), (0, 0)))
    y2p = jnp.pad(y_2, ((0, N_PAD - N), (0, 0)))
    s1, h1, lse1 = _node_stats(y1p)
    s2, h2, lse2 = _node_stats(y2p)
    pad_e = E_PAD - E
    row = jnp.concatenate(
        [edge_index[0], jnp.full((pad_e,), PAD_ROW, jnp.int32)])
    col = jnp.concatenate(
        [edge_index[1], jnp.full((pad_e,), PAD_COL, jnp.int32)])
    ewi = lax.bitcast_convert_type(
        jnp.concatenate([edge_weight, jnp.zeros((pad_e,), jnp.float32)]),
        jnp.int32)
    pk = jnp.stack([row, col, ewi]).reshape(3, NS, B_T, K).transpose(1, 2, 0, 3)
    lab = jnp.concatenate(
        [idx_label, jnp.full((L_PAD - NLAB,), PAD_LAB, jnp.int32)])
    z1, z2, cs, hs1, hs2, lc = _build_edge_stage()(
        s1, s2, h1.reshape(N_PAD), h2.reshape(N_PAD), pk, lab)
    loss = _finish(y1p, y2p, lse1, lse2, z1, z2,
                   cs.T, hs1.T, hs2.T, lc.reshape(N_PAD, 1))
    return loss.reshape(())
